# baseline (speedup 1.0000x reference)
import functools

import jax
import jax.numpy as jnp
from jax import lax
from jax.experimental import pallas as pl
from jax.experimental.pallas import tpu as pltpu

_P = 8


def _elu(x):
    return jnp.where(x > 0, x, jnp.exp(x) - 1.0)


def _shift_edges(v, lo_lanes, hi_lanes, J):
    R = v.shape[0]
    j_iota = lax.broadcasted_iota(jnp.int32, (R, 1), 0) % J
    hi = v[:, hi_lanes[0]:hi_lanes[1]]
    lo = v[:, lo_lanes[0]:lo_lanes[1]]
    zrow_h = jnp.zeros((1,) + hi.shape[1:], v.dtype)
    zrow_l = jnp.zeros((1,) + lo.shape[1:], v.dtype)
    prev_hi = jnp.concatenate([zrow_h, hi[:-1]], axis=0)
    next_lo = jnp.concatenate([lo[1:], zrow_l], axis=0)
    prev_hi = jnp.where(j_iota == 0, 0.0, prev_hi)
    next_lo = jnp.where(j_iota == J - 1, 0.0, next_lo)
    return prev_hi, next_lo


def _enc_kernel(x_ref, pp_ref, w00_ref, w0m_ref, w0p_ref, b0_ref,
                wh0_ref, whm_ref, whp_ref, bh_ref, wl_ref, bl_ref,
                y_ref, stats_ref, *, n_hidden):
    TB, L = x_ref.shape
    J = L // _P
    R = TB * J
    CpP = wl_ref.shape[0]

    a = (2.0 * x_ref[...] - 1.0).astype(jnp.bfloat16)
    planes = jnp.dot(a, pp_ref[...],
                     preferred_element_type=jnp.float32)
    xs = jnp.stack([planes[:, m * J:(m + 1) * J] for m in range(24)],
                   axis=-1).astype(jnp.bfloat16).reshape(R, 24)
    xm, xp = _shift_edges(xs, (0, 6), (18, 24), J)
    acc = jnp.dot(xs, w00_ref[...], preferred_element_type=jnp.float32)
    accL = jnp.dot(xm, w0m_ref[...], preferred_element_type=jnp.float32)
    accR = jnp.dot(xp, w0p_ref[...], preferred_element_type=jnp.float32)
    acc = jnp.concatenate(
        [acc[:, :256] + accL, acc[:, 256:768], acc[:, 768:] + accR],
        axis=1) + b0_ref[...]
    h = _elu(acc).astype(jnp.bfloat16)

    for layer in range(n_hidden):
        hm, hp = _shift_edges(h, (0, 256), (768, 1024), J)
        acc = jnp.dot(h, wh0_ref[layer], preferred_element_type=jnp.float32)
        accL = jnp.dot(hm, whm_ref[layer], preferred_element_type=jnp.float32)
        accR = jnp.dot(hp, whp_ref[layer], preferred_element_type=jnp.float32)
        acc = jnp.concatenate(
            [acc[:, :256] + accL, acc[:, 256:768], acc[:, 768:] + accR],
            axis=1) + bh_ref[layer]
        h = _elu(acc).astype(jnp.bfloat16)

    y = _elu(jnp.dot(h, wl_ref[...],
                     preferred_element_type=jnp.float32) + bl_ref[...])
    y_ref[...] = y

    zeros_t = jnp.zeros((8, 128), jnp.float32)
    stats_ref[0, 0] = zeros_t + jnp.sum(y)
    stats_ref[0, 1] = zeros_t + jnp.sum(y * y)


def _norm_kernel(scal_ref, y_ref, out_ref):
    out_ref[...] = (y_ref[...] - scal_ref[0]) * scal_ref[1]


def _pack_params(branches, c_pad):
    ks, K, C = branches[0][0].shape
    n_hidden = branches[0][2].shape[0]
    w0 = jnp.zeros((ks, 3, c_pad), jnp.float32)
    b0 = jnp.zeros((1, c_pad), jnp.float32)
    wh = jnp.zeros((n_hidden, ks, c_pad, c_pad), jnp.float32)
    bh = jnp.zeros((n_hidden, 1, c_pad), jnp.float32)
    wl = jnp.zeros((c_pad, 3), jnp.float32)
    bl = jnp.zeros((1, 3), jnp.float32)
    for r, (w0_r, b0_r, wh_r, bh_r, wl_r, bl_r) in enumerate(branches):
        w0 = w0.at[:, r, r * C:(r + 1) * C].set(w0_r[:, 0, :])
        b0 = b0.at[:, r * C:(r + 1) * C].set(b0_r)
        wh = wh.at[:, :, r * C:(r + 1) * C, r * C:(r + 1) * C].set(wh_r)
        bh = bh.at[:, 0, r * C:(r + 1) * C].set(bh_r)
        wl = wl.at[r * C:(r + 1) * C, r:r + 1].set(wl_r)
        bl = bl.at[:, r:r + 1].set(bl_r)

    P = _P
    w00 = jnp.zeros((P, 3, P, c_pad), jnp.float32)
    wh0 = jnp.zeros((n_hidden, P, c_pad, P, c_pad), jnp.float32)
    for q in range(P):
        for p in range(P):
            t = q - p + 2
            if 0 <= t < ks:
                w00 = w00.at[q, :, p, :].set(w0[t])
                wh0 = wh0.at[:, q, :, p, :].set(wh[:, t])
    w0m = jnp.zeros((2, 3, 2, c_pad), jnp.float32)
    whm = jnp.zeros((n_hidden, 2, c_pad, 2, c_pad), jnp.float32)
    w0p = jnp.zeros((2, 3, 2, c_pad), jnp.float32)
    whp = jnp.zeros((n_hidden, 2, c_pad, 2, c_pad), jnp.float32)
    for qq in range(2):
        for p in range(2):
            t = qq - p
            if 0 <= t < 2:
                w0m = w0m.at[qq, :, p, :].set(w0[t])
                whm = whm.at[:, qq, :, p, :].set(wh[:, t])
            t2 = qq + 4 - p
            if 3 <= t2 < ks:
                w0p = w0p.at[qq, :, p, :].set(w0[t2])
                whp = whp.at[:, qq, :, p, :].set(wh[:, t2])

    w00 = w00.reshape(P * 3, P * c_pad).astype(jnp.bfloat16)
    wh0 = wh0.reshape(n_hidden, P * c_pad, P * c_pad).astype(jnp.bfloat16)
    w0m = w0m.reshape(6, 2 * c_pad).astype(jnp.bfloat16)
    whm = whm.reshape(n_hidden, 2 * c_pad, 2 * c_pad).astype(jnp.bfloat16)
    w0p = w0p.reshape(6, 2 * c_pad).astype(jnp.bfloat16)
    whp = whp.reshape(n_hidden, 2 * c_pad, 2 * c_pad).astype(jnp.bfloat16)

    b8 = jnp.tile(b0, (1, P))
    bh8 = jnp.tile(bh, (1, 1, P))
    wl8 = jnp.zeros((P, c_pad, P, 3), jnp.float32)
    for p in range(P):
        wl8 = wl8.at[p, :, p, :].set(wl)
    wl8 = wl8.reshape(P * c_pad, P * 3).astype(jnp.bfloat16)
    bl8 = jnp.tile(bl, (1, P))
    return (w00, w0m, w0p, b8, wh0, whm, whp, bh8, wl8, bl8, n_hidden)


def kernel(inputs,
           b1_w0, b1_b0, b1_wh, b1_bh, b1_wl, b1_bl,
           b2_w0, b2_b0, b2_wh, b2_bh, b2_wl, b2_bl,
           b3_w0, b3_b0, b3_wh, b3_bh, b3_wl, b3_bl,
           p_array1, p_array2):
    B, L, K = inputs.shape
    P = _P
    J = L // P
    c_pad = 128
    branches = ((b1_w0, b1_b0, b1_wh, b1_bh, b1_wl, b1_bl),
                (b2_w0, b2_b0, b2_wh, b2_bh, b2_wl, b2_bl),
                (b3_w0, b3_b0, b3_wh, b3_bh, b3_wl, b3_bl))
    (w00, w0m, w0p, b8, wh0, whm, whp, bh8, wl8, bl8,
     n_hidden) = _pack_params(branches, c_pad)

    tile_b = 128
    while B % tile_b:
        tile_b -= 1
    num_tiles = B // tile_b
    R = tile_b * J

    x2 = inputs.astype(jnp.float32).reshape(B, L)
    lidx = jnp.arange(L, dtype=jnp.int32)
    perms = (lidx, p_array1, p_array2)
    cols = []
    for p in range(P):
        for br in range(3):
            cols.append(perms[br][p::P])
    src = jnp.concatenate(cols)
    pp = (lidx[:, None] == src[None, :]).astype(jnp.bfloat16)

    flops = 2 * B * (L * 24 * J + J * (24 * P * c_pad
                     + n_hidden * (P + 1) * c_pad * P * c_pad
                     + P * c_pad * 24))
    transcendentals = B * L * (c_pad * (1 + n_hidden) + 3)
    bytes_accessed = 4 * (x2.size + 2 * B * L * 3
                          + num_tiles * 2 * 8 * 128) + 2 * (
                              w00.size + wh0.size + wl8.size + pp.size)

    _fn = functools.partial(_enc_kernel, n_hidden=n_hidden)
    y2, stats = pl.pallas_call(
        _fn,
        grid=(num_tiles,),
        in_specs=[
            pl.BlockSpec((tile_b, L), lambda i: (i, 0)),
            pl.BlockSpec(pp.shape, lambda i: (0, 0)),
            pl.BlockSpec(w00.shape, lambda i: (0, 0)),
            pl.BlockSpec(w0m.shape, lambda i: (0, 0)),
            pl.BlockSpec(w0p.shape, lambda i: (0, 0)),
            pl.BlockSpec(b8.shape, lambda i: (0, 0)),
            pl.BlockSpec(wh0.shape, lambda i: (0, 0, 0)),
            pl.BlockSpec(whm.shape, lambda i: (0, 0, 0)),
            pl.BlockSpec(whp.shape, lambda i: (0, 0, 0)),
            pl.BlockSpec(bh8.shape, lambda i: (0, 0, 0)),
            pl.BlockSpec(wl8.shape, lambda i: (0, 0)),
            pl.BlockSpec(bl8.shape, lambda i: (0, 0)),
        ],
        out_shape=(
            jax.ShapeDtypeStruct((B * J, 24), jnp.float32),
            jax.ShapeDtypeStruct((num_tiles, 2, 8, 128), jnp.float32),
        ),
        out_specs=(
            pl.BlockSpec((R, 24), lambda i: (i, 0)),
            pl.BlockSpec((1, 2, 8, 128), lambda i: (i, 0, 0, 0)),
        ),
        compiler_params=pltpu.CompilerParams(
            dimension_semantics=("parallel",),
            vmem_limit_bytes=60 * 2 ** 20),
        cost_estimate=pl.CostEstimate(flops=int(flops),
                                      transcendentals=int(transcendentals),
                                      bytes_accessed=int(bytes_accessed)),
    )(x2, pp, w00, w0m, w0p, b8, wh0, whm, whp, bh8, wl8, bl8)

    n = float(B * L * 3)
    total = jnp.sum(stats[:, 0, 0, 0])
    total_sq = jnp.sum(stats[:, 1, 0, 0])
    mean = total / n
    var = (total_sq - n * mean * mean) / (n - 1.0)
    scal = jnp.stack([mean, lax.rsqrt(var)]).astype(jnp.float32)

    nflat = B * L * 3
    wide = 1024
    while nflat % wide:
        wide //= 2
    rows = nflat // wide
    yw = y2.reshape(rows, wide)
    tile_n = 1024
    while rows % tile_n:
        tile_n -= 1
    codes = pl.pallas_call(
        _norm_kernel,
        grid=(rows // tile_n,),
        in_specs=[
            pl.BlockSpec((2,), lambda i: (0,),
                         memory_space=pltpu.MemorySpace.SMEM),
            pl.BlockSpec((tile_n, wide), lambda i: (i, 0)),
        ],
        out_shape=jax.ShapeDtypeStruct((rows, wide), jnp.float32),
        out_specs=pl.BlockSpec((tile_n, wide), lambda i: (i, 0)),
        compiler_params=pltpu.CompilerParams(
            dimension_semantics=("parallel",)),
    )(scal, yw)
    return codes.reshape(B, L, 3)


# speedup vs baseline: 1.0345x; 1.0345x over previous
import functools

import jax
import jax.numpy as jnp
from jax import lax
from jax.experimental import pallas as pl
from jax.experimental.pallas import tpu as pltpu

_P = 8


def _elu(x):
    return jnp.where(x > 0, x, jnp.exp(x) - 1.0)


def _shift_edges(v, lo_lanes, hi_lanes, J):
    R = v.shape[0]
    j_iota = lax.broadcasted_iota(jnp.int32, (R, 1), 0) % J
    hi = v[:, hi_lanes[0]:hi_lanes[1]]
    lo = v[:, lo_lanes[0]:lo_lanes[1]]
    zrow_h = jnp.zeros((1,) + hi.shape[1:], v.dtype)
    zrow_l = jnp.zeros((1,) + lo.shape[1:], v.dtype)
    prev_hi = jnp.concatenate([zrow_h, hi[:-1]], axis=0)
    next_lo = jnp.concatenate([lo[1:], zrow_l], axis=0)
    prev_hi = jnp.where(j_iota == 0, 0.0, prev_hi)
    next_lo = jnp.where(j_iota == J - 1, 0.0, next_lo)
    return prev_hi, next_lo


def _enc_kernel(x_ref, pp_ref, w00_ref, w0m_ref, w0p_ref, b0_ref,
                wh0_ref, whm_ref, whp_ref, bh_ref, wl_ref, bl_ref,
                y_ref, stats_ref, *, n_hidden):
    TB, L = x_ref.shape
    J = L // _P
    R = TB * J
    CpP = wl_ref.shape[0]

    a = (2.0 * x_ref[...] - 1.0).astype(jnp.bfloat16)
    planes = jnp.dot(a, pp_ref[...],
                     preferred_element_type=jnp.float32)
    xs = jnp.stack([planes[:, m * J:(m + 1) * J] for m in range(24)],
                   axis=-1).astype(jnp.bfloat16).reshape(R, 24)
    xm, xp = _shift_edges(xs, (0, 6), (18, 24), J)
    acc = jnp.dot(xs, w00_ref[...], preferred_element_type=jnp.float32)
    accL = jnp.dot(xm, w0m_ref[...], preferred_element_type=jnp.float32)
    accR = jnp.dot(xp, w0p_ref[...], preferred_element_type=jnp.float32)
    acc = jnp.concatenate(
        [acc[:, :256] + accL, acc[:, 256:768], acc[:, 768:] + accR],
        axis=1) + b0_ref[...]
    h = _elu(acc).astype(jnp.bfloat16)

    for layer in range(n_hidden):
        hm, hp = _shift_edges(h, (0, 256), (768, 1024), J)
        acc = jnp.dot(h, wh0_ref[layer], preferred_element_type=jnp.float32)
        accL = jnp.dot(hm, whm_ref[layer], preferred_element_type=jnp.float32)
        accR = jnp.dot(hp, whp_ref[layer], preferred_element_type=jnp.float32)
        acc = jnp.concatenate(
            [acc[:, :256] + accL, acc[:, 256:768], acc[:, 768:] + accR],
            axis=1) + bh_ref[layer]
        h = _elu(acc).astype(jnp.bfloat16)

    y = _elu(jnp.dot(h, wl_ref[...],
                     preferred_element_type=jnp.float32) + bl_ref[...])
    y_ref[...] = y

    zeros_t = jnp.zeros((8, 128), jnp.float32)
    stats_ref[0, 0] = zeros_t + jnp.sum(y)
    stats_ref[0, 1] = zeros_t + jnp.sum(y * y)


def _norm_kernel(scal_ref, y_ref, out_ref):
    out_ref[...] = (y_ref[...] - scal_ref[0]) * scal_ref[1]


def _pack_params(branches, c_pad):
    ks, K, C = branches[0][0].shape
    n_hidden = branches[0][2].shape[0]
    w0 = jnp.zeros((ks, 3, c_pad), jnp.float32)
    b0 = jnp.zeros((1, c_pad), jnp.float32)
    wh = jnp.zeros((n_hidden, ks, c_pad, c_pad), jnp.float32)
    bh = jnp.zeros((n_hidden, 1, c_pad), jnp.float32)
    wl = jnp.zeros((c_pad, 3), jnp.float32)
    bl = jnp.zeros((1, 3), jnp.float32)
    for r, (w0_r, b0_r, wh_r, bh_r, wl_r, bl_r) in enumerate(branches):
        w0 = w0.at[:, r, r * C:(r + 1) * C].set(w0_r[:, 0, :])
        b0 = b0.at[:, r * C:(r + 1) * C].set(b0_r)
        wh = wh.at[:, :, r * C:(r + 1) * C, r * C:(r + 1) * C].set(wh_r)
        bh = bh.at[:, 0, r * C:(r + 1) * C].set(bh_r)
        wl = wl.at[r * C:(r + 1) * C, r:r + 1].set(wl_r)
        bl = bl.at[:, r:r + 1].set(bl_r)

    P = _P
    w00 = jnp.zeros((P, 3, P, c_pad), jnp.float32)
    wh0 = jnp.zeros((n_hidden, P, c_pad, P, c_pad), jnp.float32)
    for q in range(P):
        for p in range(P):
            t = q - p + 2
            if 0 <= t < ks:
                w00 = w00.at[q, :, p, :].set(w0[t])
                wh0 = wh0.at[:, q, :, p, :].set(wh[:, t])
    w0m = jnp.zeros((2, 3, 2, c_pad), jnp.float32)
    whm = jnp.zeros((n_hidden, 2, c_pad, 2, c_pad), jnp.float32)
    w0p = jnp.zeros((2, 3, 2, c_pad), jnp.float32)
    whp = jnp.zeros((n_hidden, 2, c_pad, 2, c_pad), jnp.float32)
    for qq in range(2):
        for p in range(2):
            t = qq - p
            if 0 <= t < 2:
                w0m = w0m.at[qq, :, p, :].set(w0[t])
                whm = whm.at[:, qq, :, p, :].set(wh[:, t])
            t2 = qq + 4 - p
            if 3 <= t2 < ks:
                w0p = w0p.at[qq, :, p, :].set(w0[t2])
                whp = whp.at[:, qq, :, p, :].set(wh[:, t2])

    w00 = w00.reshape(P * 3, P * c_pad).astype(jnp.bfloat16)
    wh0 = wh0.reshape(n_hidden, P * c_pad, P * c_pad).astype(jnp.bfloat16)
    w0m = w0m.reshape(6, 2 * c_pad).astype(jnp.bfloat16)
    whm = whm.reshape(n_hidden, 2 * c_pad, 2 * c_pad).astype(jnp.bfloat16)
    w0p = w0p.reshape(6, 2 * c_pad).astype(jnp.bfloat16)
    whp = whp.reshape(n_hidden, 2 * c_pad, 2 * c_pad).astype(jnp.bfloat16)

    b8 = jnp.tile(b0, (1, P))
    bh8 = jnp.tile(bh, (1, 1, P))
    wl8 = jnp.zeros((P, c_pad, P, 3), jnp.float32)
    for p in range(P):
        wl8 = wl8.at[p, :, p, :].set(wl)
    wl8 = wl8.reshape(P * c_pad, P * 3).astype(jnp.bfloat16)
    bl8 = jnp.tile(bl, (1, P))
    return (w00, w0m, w0p, b8, wh0, whm, whp, bh8, wl8, bl8, n_hidden)


def kernel(inputs,
           b1_w0, b1_b0, b1_wh, b1_bh, b1_wl, b1_bl,
           b2_w0, b2_b0, b2_wh, b2_bh, b2_wl, b2_bl,
           b3_w0, b3_b0, b3_wh, b3_bh, b3_wl, b3_bl,
           p_array1, p_array2):
    B, L, K = inputs.shape
    P = _P
    J = L // P
    c_pad = 128
    branches = ((b1_w0, b1_b0, b1_wh, b1_bh, b1_wl, b1_bl),
                (b2_w0, b2_b0, b2_wh, b2_bh, b2_wl, b2_bl),
                (b3_w0, b3_b0, b3_wh, b3_bh, b3_wl, b3_bl))
    (w00, w0m, w0p, b8, wh0, whm, whp, bh8, wl8, bl8,
     n_hidden) = _pack_params(branches, c_pad)

    tile_b = 64
    while B % tile_b:
        tile_b -= 1
    num_tiles = B // tile_b
    R = tile_b * J

    x2 = inputs.astype(jnp.float32).reshape(B, L)
    lidx = jnp.arange(L, dtype=jnp.int32)
    perms = (lidx, p_array1, p_array2)
    cols = []
    for p in range(P):
        for br in range(3):
            cols.append(perms[br][p::P])
    src = jnp.concatenate(cols)
    pp = (lidx[:, None] == src[None, :]).astype(jnp.bfloat16)

    flops = 2 * B * (L * 24 * J + J * (24 * P * c_pad
                     + n_hidden * (P + 1) * c_pad * P * c_pad
                     + P * c_pad * 24))
    transcendentals = B * L * (c_pad * (1 + n_hidden) + 3)
    bytes_accessed = 4 * (x2.size + 2 * B * L * 3
                          + num_tiles * 2 * 8 * 128) + 2 * (
                              w00.size + wh0.size + wl8.size + pp.size)

    _fn = functools.partial(_enc_kernel, n_hidden=n_hidden)
    y2, stats = pl.pallas_call(
        _fn,
        grid=(num_tiles,),
        in_specs=[
            pl.BlockSpec((tile_b, L), lambda i: (i, 0)),
            pl.BlockSpec(pp.shape, lambda i: (0, 0)),
            pl.BlockSpec(w00.shape, lambda i: (0, 0)),
            pl.BlockSpec(w0m.shape, lambda i: (0, 0)),
            pl.BlockSpec(w0p.shape, lambda i: (0, 0)),
            pl.BlockSpec(b8.shape, lambda i: (0, 0)),
            pl.BlockSpec(wh0.shape, lambda i: (0, 0, 0)),
            pl.BlockSpec(whm.shape, lambda i: (0, 0, 0)),
            pl.BlockSpec(whp.shape, lambda i: (0, 0, 0)),
            pl.BlockSpec(bh8.shape, lambda i: (0, 0, 0)),
            pl.BlockSpec(wl8.shape, lambda i: (0, 0)),
            pl.BlockSpec(bl8.shape, lambda i: (0, 0)),
        ],
        out_shape=(
            jax.ShapeDtypeStruct((B * J, 24), jnp.float32),
            jax.ShapeDtypeStruct((num_tiles, 2, 8, 128), jnp.float32),
        ),
        out_specs=(
            pl.BlockSpec((R, 24), lambda i: (i, 0)),
            pl.BlockSpec((1, 2, 8, 128), lambda i: (i, 0, 0, 0)),
        ),
        compiler_params=pltpu.CompilerParams(
            dimension_semantics=("parallel",),
            vmem_limit_bytes=60 * 2 ** 20),
        cost_estimate=pl.CostEstimate(flops=int(flops),
                                      transcendentals=int(transcendentals),
                                      bytes_accessed=int(bytes_accessed)),
    )(x2, pp, w00, w0m, w0p, b8, wh0, whm, whp, bh8, wl8, bl8)

    n = float(B * L * 3)
    total = jnp.sum(stats[:, 0, 0, 0])
    total_sq = jnp.sum(stats[:, 1, 0, 0])
    mean = total / n
    var = (total_sq - n * mean * mean) / (n - 1.0)
    scal = jnp.stack([mean, lax.rsqrt(var)]).astype(jnp.float32)

    nflat = B * L * 3
    wide = 1024
    while nflat % wide:
        wide //= 2
    rows = nflat // wide
    yw = y2.reshape(rows, wide)
    tile_n = 1024
    while rows % tile_n:
        tile_n -= 1
    codes = pl.pallas_call(
        _norm_kernel,
        grid=(rows // tile_n,),
        in_specs=[
            pl.BlockSpec((2,), lambda i: (0,),
                         memory_space=pltpu.MemorySpace.SMEM),
            pl.BlockSpec((tile_n, wide), lambda i: (i, 0)),
        ],
        out_shape=jax.ShapeDtypeStruct((rows, wide), jnp.float32),
        out_specs=pl.BlockSpec((tile_n, wide), lambda i: (i, 0)),
        compiler_params=pltpu.CompilerParams(
            dimension_semantics=("parallel",)),
    )(scal, yw)
    return codes.reshape(B, L, 3)


# speedup vs baseline: 1.3232x; 1.2791x over previous
import functools

import jax
import jax.numpy as jnp
from jax import lax
from jax.experimental import pallas as pl
from jax.experimental.pallas import tpu as pltpu

_P = 8


def _elu(x):
    return jnp.where(x > 0, x, jnp.exp(x) - 1.0)


def _shift_edges(v, lo_lanes, hi_lanes, J):
    R = v.shape[0]
    j_iota = lax.broadcasted_iota(jnp.int32, (R, 1), 0) % J
    hi = v[:, hi_lanes[0]:hi_lanes[1]]
    lo = v[:, lo_lanes[0]:lo_lanes[1]]
    zrow_h = jnp.zeros((1,) + hi.shape[1:], v.dtype)
    zrow_l = jnp.zeros((1,) + lo.shape[1:], v.dtype)
    prev_hi = jnp.concatenate([zrow_h, hi[:-1]], axis=0)
    next_lo = jnp.concatenate([lo[1:], zrow_l], axis=0)
    prev_hi = jnp.where(j_iota == 0, 0.0, prev_hi)
    next_lo = jnp.where(j_iota == J - 1, 0.0, next_lo)
    return prev_hi, next_lo


def _enc_kernel(x_ref, pp_ref, w00_ref, w0m_ref, w0p_ref, b0_ref,
                wh0_ref, whm_ref, whp_ref, bh_ref, wl_ref, bl_ref,
                y_ref, stats_ref, *, n_hidden):
    TB, L = x_ref.shape
    J = L // _P
    R = TB * J
    CpP = wl_ref.shape[0]

    a = (2.0 * x_ref[...] - 1.0).astype(jnp.bfloat16)
    planes = jnp.dot(a, pp_ref[...],
                     preferred_element_type=jnp.float32).astype(jnp.bfloat16)
    xs = jnp.stack([planes[:, m * J:(m + 1) * J] for m in range(24)],
                   axis=-1).reshape(R, 24)
    xm, xp = _shift_edges(xs, (0, 6), (18, 24), J)
    acc = jnp.dot(xs, w00_ref[...], preferred_element_type=jnp.float32)
    accL = jnp.dot(xm, w0m_ref[...], preferred_element_type=jnp.float32)
    accR = jnp.dot(xp, w0p_ref[...], preferred_element_type=jnp.float32)
    acc = jnp.concatenate(
        [acc[:, :256] + accL, acc[:, 256:768], acc[:, 768:] + accR],
        axis=1) + b0_ref[...]
    h = _elu(acc).astype(jnp.bfloat16)

    for layer in range(n_hidden):
        hm, hp = _shift_edges(h, (0, 256), (768, 1024), J)
        parts = []
        for g in range(4):
            k0 = max(0, 2 * g - 2) * 128
            k1 = min(8, 2 * g + 4) * 128
            parts.append(jnp.dot(
                h[:, k0:k1], wh0_ref[layer, k0:k1, 256 * g:256 * (g + 1)],
                preferred_element_type=jnp.float32))
        accL = jnp.dot(hm, whm_ref[layer], preferred_element_type=jnp.float32)
        accR = jnp.dot(hp, whp_ref[layer], preferred_element_type=jnp.float32)
        acc = jnp.concatenate(
            [parts[0] + accL, parts[1], parts[2], parts[3] + accR],
            axis=1) + bh_ref[layer]
        h = _elu(acc).astype(jnp.bfloat16)

    y = _elu(jnp.dot(h, wl_ref[...],
                     preferred_element_type=jnp.float32) + bl_ref[...])
    y_ref[...] = y

    zeros_t = jnp.zeros((8, 128), jnp.float32)
    stats_ref[0, 0] = zeros_t + jnp.sum(y)
    stats_ref[0, 1] = zeros_t + jnp.sum(y * y)


def _norm_kernel(scal_ref, y_ref, out_ref):
    out_ref[...] = (y_ref[...] - scal_ref[0]) * scal_ref[1]


def _pack_params(branches, c_pad):
    ks, K, C = branches[0][0].shape
    n_hidden = branches[0][2].shape[0]
    w0 = jnp.zeros((ks, 3, c_pad), jnp.float32)
    b0 = jnp.zeros((1, c_pad), jnp.float32)
    wh = jnp.zeros((n_hidden, ks, c_pad, c_pad), jnp.float32)
    bh = jnp.zeros((n_hidden, 1, c_pad), jnp.float32)
    wl = jnp.zeros((c_pad, 3), jnp.float32)
    bl = jnp.zeros((1, 3), jnp.float32)
    for r, (w0_r, b0_r, wh_r, bh_r, wl_r, bl_r) in enumerate(branches):
        w0 = w0.at[:, r, r * C:(r + 1) * C].set(w0_r[:, 0, :])
        b0 = b0.at[:, r * C:(r + 1) * C].set(b0_r)
        wh = wh.at[:, :, r * C:(r + 1) * C, r * C:(r + 1) * C].set(wh_r)
        bh = bh.at[:, 0, r * C:(r + 1) * C].set(bh_r)
        wl = wl.at[r * C:(r + 1) * C, r:r + 1].set(wl_r)
        bl = bl.at[:, r:r + 1].set(bl_r)

    P = _P
    w0b = w0.astype(jnp.bfloat16)
    whb = wh.astype(jnp.bfloat16)
    z0 = jnp.zeros((3, c_pad), jnp.bfloat16)
    zh = jnp.zeros((n_hidden, c_pad, c_pad), jnp.bfloat16)
    w00 = jnp.concatenate([
        jnp.concatenate([w0b[q - p + 2] if 0 <= q - p + 2 < ks else z0
                         for p in range(P)], axis=1)
        for q in range(P)], axis=0)
    wh0 = jnp.concatenate([
        jnp.concatenate([whb[:, q - p + 2] if 0 <= q - p + 2 < ks else zh
                         for p in range(P)], axis=2)
        for q in range(P)], axis=1)
    w0m = jnp.concatenate([
        jnp.concatenate([w0b[qq - p] if 0 <= qq - p < 2 else z0
                         for p in range(2)], axis=1)
        for qq in range(2)], axis=0)
    whm = jnp.concatenate([
        jnp.concatenate([whb[:, qq - p] if 0 <= qq - p < 2 else zh
                         for p in range(2)], axis=2)
        for qq in range(2)], axis=1)
    w0p = jnp.concatenate([
        jnp.concatenate([w0b[qq + 4 - p] if 3 <= qq + 4 - p < ks else z0
                         for p in range(2)], axis=1)
        for qq in range(2)], axis=0)
    whp = jnp.concatenate([
        jnp.concatenate([whb[:, qq + 4 - p] if 3 <= qq + 4 - p < ks else zh
                         for p in range(2)], axis=2)
        for qq in range(2)], axis=1)

    b8 = jnp.tile(b0, (1, P))
    bh8 = jnp.tile(bh, (1, 1, P))
    wlb = wl.astype(jnp.bfloat16)
    zl = jnp.zeros((c_pad, 3), jnp.bfloat16)
    wl8 = jnp.concatenate([
        jnp.concatenate([wlb if p == q else zl for p in range(P)], axis=1)
        for q in range(P)], axis=0)
    bl8 = jnp.tile(bl, (1, P))
    return (w00, w0m, w0p, b8, wh0, whm, whp, bh8, wl8, bl8, n_hidden)


def kernel(inputs,
           b1_w0, b1_b0, b1_wh, b1_bh, b1_wl, b1_bl,
           b2_w0, b2_b0, b2_wh, b2_bh, b2_wl, b2_bl,
           b3_w0, b3_b0, b3_wh, b3_bh, b3_wl, b3_bl,
           p_array1, p_array2):
    B, L, K = inputs.shape
    P = _P
    J = L // P
    c_pad = 128
    branches = ((b1_w0, b1_b0, b1_wh, b1_bh, b1_wl, b1_bl),
                (b2_w0, b2_b0, b2_wh, b2_bh, b2_wl, b2_bl),
                (b3_w0, b3_b0, b3_wh, b3_bh, b3_wl, b3_bl))
    (w00, w0m, w0p, b8, wh0, whm, whp, bh8, wl8, bl8,
     n_hidden) = _pack_params(branches, c_pad)

    tile_b = 64
    while B % tile_b:
        tile_b -= 1
    num_tiles = B // tile_b
    R = tile_b * J

    x2 = inputs.astype(jnp.float32).reshape(B, L)
    lidx = jnp.arange(L, dtype=jnp.int32)
    perms = (lidx, p_array1, p_array2)
    cols = []
    for p in range(P):
        for br in range(3):
            cols.append(perms[br][p::P])
    src = jnp.concatenate(cols)
    pp = (lidx[:, None] == src[None, :]).astype(jnp.bfloat16)

    flops = 2 * B * (L * 24 * J + J * (24 * P * c_pad
                     + n_hidden * (P + 1) * c_pad * P * c_pad
                     + P * c_pad * 24))
    transcendentals = B * L * (c_pad * (1 + n_hidden) + 3)
    bytes_accessed = 4 * (x2.size + 2 * B * L * 3
                          + num_tiles * 2 * 8 * 128) + 2 * (
                              w00.size + wh0.size + wl8.size + pp.size)

    _fn = functools.partial(_enc_kernel, n_hidden=n_hidden)
    y2, stats = pl.pallas_call(
        _fn,
        grid=(num_tiles,),
        in_specs=[
            pl.BlockSpec((tile_b, L), lambda i: (i, 0)),
            pl.BlockSpec(pp.shape, lambda i: (0, 0)),
            pl.BlockSpec(w00.shape, lambda i: (0, 0)),
            pl.BlockSpec(w0m.shape, lambda i: (0, 0)),
            pl.BlockSpec(w0p.shape, lambda i: (0, 0)),
            pl.BlockSpec(b8.shape, lambda i: (0, 0)),
            pl.BlockSpec(wh0.shape, lambda i: (0, 0, 0)),
            pl.BlockSpec(whm.shape, lambda i: (0, 0, 0)),
            pl.BlockSpec(whp.shape, lambda i: (0, 0, 0)),
            pl.BlockSpec(bh8.shape, lambda i: (0, 0, 0)),
            pl.BlockSpec(wl8.shape, lambda i: (0, 0)),
            pl.BlockSpec(bl8.shape, lambda i: (0, 0)),
        ],
        out_shape=(
            jax.ShapeDtypeStruct((B * J, 24), jnp.float32),
            jax.ShapeDtypeStruct((num_tiles, 2, 8, 128), jnp.float32),
        ),
        out_specs=(
            pl.BlockSpec((R, 24), lambda i: (i, 0)),
            pl.BlockSpec((1, 2, 8, 128), lambda i: (i, 0, 0, 0)),
        ),
        compiler_params=pltpu.CompilerParams(
            dimension_semantics=("parallel",),
            vmem_limit_bytes=60 * 2 ** 20),
        cost_estimate=pl.CostEstimate(flops=int(flops),
                                      transcendentals=int(transcendentals),
                                      bytes_accessed=int(bytes_accessed)),
    )(x2, pp, w00, w0m, w0p, b8, wh0, whm, whp, bh8, wl8, bl8)

    n = float(B * L * 3)
    total = jnp.sum(stats[:, 0, 0, 0])
    total_sq = jnp.sum(stats[:, 1, 0, 0])
    mean = total / n
    var = (total_sq - n * mean * mean) / (n - 1.0)
    scal = jnp.stack([mean, lax.rsqrt(var)]).astype(jnp.float32)

    nflat = B * L * 3
    wide = 1024
    while nflat % wide:
        wide //= 2
    rows = nflat // wide
    yw = y2.reshape(rows, wide)
    tile_n = 1024
    while rows % tile_n:
        tile_n -= 1
    codes = pl.pallas_call(
        _norm_kernel,
        grid=(rows // tile_n,),
        in_specs=[
            pl.BlockSpec((2,), lambda i: (0,),
                         memory_space=pltpu.MemorySpace.SMEM),
            pl.BlockSpec((tile_n, wide), lambda i: (i, 0)),
        ],
        out_shape=jax.ShapeDtypeStruct((rows, wide), jnp.float32),
        out_specs=pl.BlockSpec((tile_n, wide), lambda i: (i, 0)),
        compiler_params=pltpu.CompilerParams(
            dimension_semantics=("parallel",)),
    )(scal, yw)
    return codes.reshape(B, L, 3)


# speedup vs baseline: 1.3702x; 1.0355x over previous
import functools

import jax
import jax.numpy as jnp
from jax import lax
from jax.experimental import pallas as pl
from jax.experimental.pallas import tpu as pltpu

_P = 8


def _elu(x):
    return jnp.where(x > 0, x, jnp.exp(x) - 1.0)


def _shift_edges(v, lo_lanes, hi_lanes, J):
    R = v.shape[0]
    j_iota = lax.broadcasted_iota(jnp.int32, (R, 1), 0) % J
    hi = v[:, hi_lanes[0]:hi_lanes[1]]
    lo = v[:, lo_lanes[0]:lo_lanes[1]]
    zrow_h = jnp.zeros((1,) + hi.shape[1:], v.dtype)
    zrow_l = jnp.zeros((1,) + lo.shape[1:], v.dtype)
    prev_hi = jnp.concatenate([zrow_h, hi[:-1]], axis=0)
    next_lo = jnp.concatenate([lo[1:], zrow_l], axis=0)
    prev_hi = jnp.where(j_iota == 0, 0.0, prev_hi)
    next_lo = jnp.where(j_iota == J - 1, 0.0, next_lo)
    return prev_hi, next_lo


def _enc_kernel(x_ref, pp_ref, w00_ref, w0m_ref, w0p_ref, b0_ref,
                wh0_ref, whm_ref, whp_ref, bh_ref, wl_ref, bl_ref,
                y_ref, stats_ref, *, n_hidden):
    TB, L = x_ref.shape
    J = L // _P
    R = TB * J
    CpP = wl_ref.shape[0]

    a = (2.0 * x_ref[...] - 1.0).astype(jnp.bfloat16)
    planes = jnp.dot(a, pp_ref[...],
                     preferred_element_type=jnp.float32).astype(jnp.bfloat16)
    xs = jnp.stack([planes[:, m * J:(m + 1) * J] for m in range(24)],
                   axis=-1).reshape(R, 24)
    xm, xp = _shift_edges(xs, (0, 6), (18, 24), J)
    acc = jnp.dot(xs, w00_ref[...], preferred_element_type=jnp.float32)
    accL = jnp.dot(xm, w0m_ref[...], preferred_element_type=jnp.float32)
    accR = jnp.dot(xp, w0p_ref[...], preferred_element_type=jnp.float32)
    acc = jnp.concatenate(
        [acc[:, :256] + accL, acc[:, 256:768], acc[:, 768:] + accR],
        axis=1) + b0_ref[...]
    h = _elu(acc.astype(jnp.bfloat16))

    for layer in range(n_hidden):
        hm, hp = _shift_edges(h, (0, 256), (768, 1024), J)
        parts = []
        for g in range(4):
            k0 = max(0, 2 * g - 2) * 128
            k1 = min(8, 2 * g + 4) * 128
            parts.append(jnp.dot(
                h[:, k0:k1], wh0_ref[layer, k0:k1, 256 * g:256 * (g + 1)],
                preferred_element_type=jnp.float32))
        accL = jnp.dot(hm, whm_ref[layer], preferred_element_type=jnp.float32)
        accR = jnp.dot(hp, whp_ref[layer], preferred_element_type=jnp.float32)
        acc = jnp.concatenate(
            [parts[0] + accL, parts[1], parts[2], parts[3] + accR],
            axis=1) + bh_ref[layer]
        h = _elu(acc.astype(jnp.bfloat16))

    y = _elu(jnp.dot(h, wl_ref[...],
                     preferred_element_type=jnp.float32) + bl_ref[...])
    y_ref[...] = y

    zeros_t = jnp.zeros((8, 128), jnp.float32)
    stats_ref[0, 0] = zeros_t + jnp.sum(y)
    stats_ref[0, 1] = zeros_t + jnp.sum(y * y)


def _norm_kernel(scal_ref, y_ref, out_ref):
    out_ref[...] = (y_ref[...] - scal_ref[0]) * scal_ref[1]


def _pack_params(branches, c_pad):
    ks, K, C = branches[0][0].shape
    n_hidden = branches[0][2].shape[0]
    w0 = jnp.zeros((ks, 3, c_pad), jnp.float32)
    b0 = jnp.zeros((1, c_pad), jnp.float32)
    wh = jnp.zeros((n_hidden, ks, c_pad, c_pad), jnp.float32)
    bh = jnp.zeros((n_hidden, 1, c_pad), jnp.float32)
    wl = jnp.zeros((c_pad, 3), jnp.float32)
    bl = jnp.zeros((1, 3), jnp.float32)
    for r, (w0_r, b0_r, wh_r, bh_r, wl_r, bl_r) in enumerate(branches):
        w0 = w0.at[:, r, r * C:(r + 1) * C].set(w0_r[:, 0, :])
        b0 = b0.at[:, r * C:(r + 1) * C].set(b0_r)
        wh = wh.at[:, :, r * C:(r + 1) * C, r * C:(r + 1) * C].set(wh_r)
        bh = bh.at[:, 0, r * C:(r + 1) * C].set(bh_r)
        wl = wl.at[r * C:(r + 1) * C, r:r + 1].set(wl_r)
        bl = bl.at[:, r:r + 1].set(bl_r)

    P = _P
    w0b = w0.astype(jnp.bfloat16)
    whb = wh.astype(jnp.bfloat16)
    z0 = jnp.zeros((3, c_pad), jnp.bfloat16)
    zh = jnp.zeros((n_hidden, c_pad, c_pad), jnp.bfloat16)
    w00 = jnp.concatenate([
        jnp.concatenate([w0b[q - p + 2] if 0 <= q - p + 2 < ks else z0
                         for p in range(P)], axis=1)
        for q in range(P)], axis=0)
    wh0 = jnp.concatenate([
        jnp.concatenate([whb[:, q - p + 2] if 0 <= q - p + 2 < ks else zh
                         for p in range(P)], axis=2)
        for q in range(P)], axis=1)
    w0m = jnp.concatenate([
        jnp.concatenate([w0b[qq - p] if 0 <= qq - p < 2 else z0
                         for p in range(2)], axis=1)
        for qq in range(2)], axis=0)
    whm = jnp.concatenate([
        jnp.concatenate([whb[:, qq - p] if 0 <= qq - p < 2 else zh
                         for p in range(2)], axis=2)
        for qq in range(2)], axis=1)
    w0p = jnp.concatenate([
        jnp.concatenate([w0b[qq + 4 - p] if 3 <= qq + 4 - p < ks else z0
                         for p in range(2)], axis=1)
        for qq in range(2)], axis=0)
    whp = jnp.concatenate([
        jnp.concatenate([whb[:, qq + 4 - p] if 3 <= qq + 4 - p < ks else zh
                         for p in range(2)], axis=2)
        for qq in range(2)], axis=1)

    b8 = jnp.tile(b0, (1, P))
    bh8 = jnp.tile(bh, (1, 1, P))
    wlb = wl.astype(jnp.bfloat16)
    zl = jnp.zeros((c_pad, 3), jnp.bfloat16)
    wl8 = jnp.concatenate([
        jnp.concatenate([wlb if p == q else zl for p in range(P)], axis=1)
        for q in range(P)], axis=0)
    bl8 = jnp.tile(bl, (1, P))
    return (w00, w0m, w0p, b8, wh0, whm, whp, bh8, wl8, bl8, n_hidden)


def kernel(inputs,
           b1_w0, b1_b0, b1_wh, b1_bh, b1_wl, b1_bl,
           b2_w0, b2_b0, b2_wh, b2_bh, b2_wl, b2_bl,
           b3_w0, b3_b0, b3_wh, b3_bh, b3_wl, b3_bl,
           p_array1, p_array2):
    B, L, K = inputs.shape
    P = _P
    J = L // P
    c_pad = 128
    branches = ((b1_w0, b1_b0, b1_wh, b1_bh, b1_wl, b1_bl),
                (b2_w0, b2_b0, b2_wh, b2_bh, b2_wl, b2_bl),
                (b3_w0, b3_b0, b3_wh, b3_bh, b3_wl, b3_bl))
    (w00, w0m, w0p, b8, wh0, whm, whp, bh8, wl8, bl8,
     n_hidden) = _pack_params(branches, c_pad)

    tile_b = 64
    while B % tile_b:
        tile_b -= 1
    num_tiles = B // tile_b
    R = tile_b * J

    x2 = inputs.astype(jnp.float32).reshape(B, L)
    lidx = jnp.arange(L, dtype=jnp.int32)
    perms = (lidx, p_array1, p_array2)
    cols = []
    for p in range(P):
        for br in range(3):
            cols.append(perms[br][p::P])
    src = jnp.concatenate(cols)
    pp = (lidx[:, None] == src[None, :]).astype(jnp.bfloat16)

    flops = 2 * B * (L * 24 * J + J * (24 * P * c_pad
                     + n_hidden * (P + 1) * c_pad * P * c_pad
                     + P * c_pad * 24))
    transcendentals = B * L * (c_pad * (1 + n_hidden) + 3)
    bytes_accessed = 4 * (x2.size + 2 * B * L * 3
                          + num_tiles * 2 * 8 * 128) + 2 * (
                              w00.size + wh0.size + wl8.size + pp.size)

    _fn = functools.partial(_enc_kernel, n_hidden=n_hidden)
    y2, stats = pl.pallas_call(
        _fn,
        grid=(num_tiles,),
        in_specs=[
            pl.BlockSpec((tile_b, L), lambda i: (i, 0)),
            pl.BlockSpec(pp.shape, lambda i: (0, 0)),
            pl.BlockSpec(w00.shape, lambda i: (0, 0)),
            pl.BlockSpec(w0m.shape, lambda i: (0, 0)),
            pl.BlockSpec(w0p.shape, lambda i: (0, 0)),
            pl.BlockSpec(b8.shape, lambda i: (0, 0)),
            pl.BlockSpec(wh0.shape, lambda i: (0, 0, 0)),
            pl.BlockSpec(whm.shape, lambda i: (0, 0, 0)),
            pl.BlockSpec(whp.shape, lambda i: (0, 0, 0)),
            pl.BlockSpec(bh8.shape, lambda i: (0, 0, 0)),
            pl.BlockSpec(wl8.shape, lambda i: (0, 0)),
            pl.BlockSpec(bl8.shape, lambda i: (0, 0)),
        ],
        out_shape=(
            jax.ShapeDtypeStruct((B * J, 24), jnp.float32),
            jax.ShapeDtypeStruct((num_tiles, 2, 8, 128), jnp.float32),
        ),
        out_specs=(
            pl.BlockSpec((R, 24), lambda i: (i, 0)),
            pl.BlockSpec((1, 2, 8, 128), lambda i: (i, 0, 0, 0)),
        ),
        compiler_params=pltpu.CompilerParams(
            dimension_semantics=("parallel",),
            vmem_limit_bytes=60 * 2 ** 20),
        cost_estimate=pl.CostEstimate(flops=int(flops),
                                      transcendentals=int(transcendentals),
                                      bytes_accessed=int(bytes_accessed)),
    )(x2, pp, w00, w0m, w0p, b8, wh0, whm, whp, bh8, wl8, bl8)

    n = float(B * L * 3)
    total = jnp.sum(stats[:, 0, 0, 0])
    total_sq = jnp.sum(stats[:, 1, 0, 0])
    mean = total / n
    var = (total_sq - n * mean * mean) / (n - 1.0)
    scal = jnp.stack([mean, lax.rsqrt(var)]).astype(jnp.float32)

    nflat = B * L * 3
    wide = 1024
    while nflat % wide:
        wide //= 2
    rows = nflat // wide
    yw = y2.reshape(rows, wide)
    tile_n = 1024
    while rows % tile_n:
        tile_n -= 1
    codes = pl.pallas_call(
        _norm_kernel,
        grid=(rows // tile_n,),
        in_specs=[
            pl.BlockSpec((2,), lambda i: (0,),
                         memory_space=pltpu.MemorySpace.SMEM),
            pl.BlockSpec((tile_n, wide), lambda i: (i, 0)),
        ],
        out_shape=jax.ShapeDtypeStruct((rows, wide), jnp.float32),
        out_specs=pl.BlockSpec((tile_n, wide), lambda i: (i, 0)),
        compiler_params=pltpu.CompilerParams(
            dimension_semantics=("parallel",)),
    )(scal, yw)
    return codes.reshape(B, L, 3)
